# Initial kernel scaffold; baseline (speedup 1.0000x reference)
import jax
import jax.numpy as jnp
from jax.experimental import pallas as pl


def kernel(x, neighbors, kernel):
    raise NotImplementedError("write your pallas kernel here")


# speedup vs baseline: 21.0577x; 21.0577x over previous
import functools

import jax
import jax.numpy as jnp
from jax import lax
from jax.experimental import pallas as pl
from jax.experimental.pallas import tpu as pltpu
from jax.experimental.pallas import tpu_sc as plsc

N = 10000
K = 32
DF = 128
U = 128

E = N * K
CH = 128
NCHUNKS = E // CH


def _matmul_body(x_ref, w_ref, o_ref):
    o_ref[...] = jnp.dot(x_ref[...], w_ref[...],
                         preferred_element_type=jnp.float32)


def _matmul(x, w):
    B = 2000
    return pl.pallas_call(
        _matmul_body,
        grid=(N // B,),
        in_specs=[
            pl.BlockSpec((B, DF), lambda i: (i, 0)),
            pl.BlockSpec((DF, U), lambda i: (0, 0)),
        ],
        out_specs=pl.BlockSpec((B, U), lambda i: (i, 0)),
        out_shape=jax.ShapeDtypeStruct((N, U), jnp.float32),
    )(x, w)


def _sc_gather(table, idx):
    info = plsc.get_sparse_core_info()
    nc, ns = info.num_cores, info.num_subcores
    nw = nc * ns
    mesh = plsc.VectorSubcoreMesh(core_axis_name="c", subcore_axis_name="s")

    @functools.partial(
        pl.kernel,
        mesh=mesh,
        out_type=jax.ShapeDtypeStruct((E, U), jnp.float32),
        scratch_types=[
            pltpu.VMEM((CH,), jnp.int32),
            pltpu.VMEM((CH, U), jnp.float32),
            pltpu.SemaphoreType.DMA,
        ],
    )
    def gk(table_hbm, idx_hbm, out_hbm, idx_v, rows_v, sem):
        wid = lax.axis_index("s") * nc + lax.axis_index("c")
        trips = (NCHUNKS - wid + nw - 1) // nw

        def body(t, carry):
            off = (wid + t * nw) * CH
            pltpu.sync_copy(idx_hbm.at[pl.ds(off, CH)], idx_v)
            pltpu.async_copy(table_hbm.at[idx_v], rows_v, sem).wait()
            pltpu.sync_copy(rows_v, out_hbm.at[pl.ds(off, CH)])
            return carry

        lax.fori_loop(0, trips, body, 0)

    return gk(table, idx)


def _batcher_pairs(n):
    pairs = []
    p = 1
    while p < n:
        k = p
        while k >= 1:
            for j in range(k % p, n - k, 2 * k):
                for i in range(min(k, n - j - k)):
                    if (i + j) // (2 * p) == (i + j + k) // (2 * p):
                        pairs.append((i + j, i + j + k))
            k //= 2
        p *= 2
    return pairs


_PAIRS16 = _batcher_pairs(16)


def _sort16(vals):
    vals = list(vals)
    for a, b in _PAIRS16:
        lo = jnp.minimum(vals[a], vals[b])
        hi = jnp.maximum(vals[a], vals[b])
        vals[a], vals[b] = lo, hi
    return vals


def _median32(vals):
    a = _sort16(vals[:16])
    b = _sort16(vals[16:])
    lo = [jnp.minimum(a[i], b[15 - i]) for i in range(16)]
    hi = [jnp.maximum(a[i], b[15 - i]) for i in range(16)]
    mx = functools.reduce(jnp.maximum, lo)
    mn = functools.reduce(jnp.minimum, hi)
    return (mx + mn) * 0.5


def _median_body(msg_ref, o_ref):
    vals = [msg_ref[k] for k in range(K)]
    o_ref[...] = _median32(vals)


def _median(msg):
    B = 200
    return pl.pallas_call(
        _median_body,
        grid=(N // B,),
        in_specs=[pl.BlockSpec((K, B, U), lambda i: (0, i, 0))],
        out_specs=pl.BlockSpec((B, U), lambda i: (i, 0)),
        out_shape=jax.ShapeDtypeStruct((N, U), jnp.float32),
    )(msg)


def kernel(x, neighbors, kernel):
    w = kernel
    h = _matmul(x, w)
    idx = neighbors.astype(jnp.int32).T.reshape(-1)
    msg = _sc_gather(h, idx)
    return _median(msg.reshape(K, N, U))
